# Initial kernel scaffold; baseline (speedup 1.0000x reference)
import jax
import jax.numpy as jnp
from jax.experimental import pallas as pl


def kernel(x, W, b, step_embed, step):
    raise NotImplementedError("write your pallas kernel here")


# speedup vs baseline: 1.2654x; 1.2654x over previous
import functools

import jax
import jax.numpy as jnp
from jax.experimental import pallas as pl
from jax.experimental.pallas import tpu as pltpu

_B, _S, _H = 4, 4096, 2048
_N = _B * _S
_K = 8192
_ALPHA = 0.01
_BLK = 1024
_R = 128


def _score_body(x_ref, w_ref, c_ref, g_ref, s_ref):
    s = jax.lax.dot_general(
        x_ref[...], w_ref[...], (((1,), (0,)), ((), ())),
        preferred_element_type=jnp.float32)
    s = s + c_ref[0, 0]
    s_ref[...] = s
    g_ref[...] = jax.nn.sigmoid(s)


def _select_body(s_ref, mask_ref, aux_ref):
    s = s_ref[...]
    i = pltpu.bitcast(s, jnp.int32)
    u = pltpu.bitcast(
        jnp.where(i >= 0, i | jnp.int32(-2147483648), ~i), jnp.uint32)

    def bit_step(t, res):
        trial = res | (jnp.uint32(1) << jnp.uint32(31 - t))
        cnt = jnp.sum((u >= trial).astype(jnp.int32))
        return jnp.where(cnt >= _K, trial, res)

    kth = jax.lax.fori_loop(0, 32, bit_step, jnp.uint32(0))

    gt = u > kth
    eq = u == kth
    n_gt = jnp.sum(gt.astype(jnp.int32))
    need = _K - n_gt

    eqf = eq.astype(jnp.float32)
    rows = jax.lax.broadcasted_iota(jnp.int32, (_R, _R), 0)
    cols = jax.lax.broadcasted_iota(jnp.int32, (_R, _R), 1)
    up_strict = (rows < cols).astype(jnp.float32)
    lo_strict = (rows > cols).astype(jnp.float32)
    in_row = jax.lax.dot_general(
        eqf, up_strict, (((1,), (0,)), ((), ())),
        preferred_element_type=jnp.float32)
    row_tot = jnp.sum(eqf, axis=1, keepdims=True)
    row_off = jax.lax.dot_general(
        lo_strict, row_tot, (((1,), (0,)), ((), ())),
        preferred_element_type=jnp.float32)
    rank = in_row + row_off
    keep_tie = eq & (rank < need.astype(jnp.float32))

    mask = jnp.where(gt | keep_tie, 1.0, 0.0).astype(jnp.float32)
    mask_ref[...] = mask
    used = jnp.sum(mask) * (1.0 / _N)
    aux_ref[...] = jnp.full((1, 1), _ALPHA * (used - (_K / _N)) ** 2,
                            dtype=jnp.float32)


@jax.jit
def _run(x, W, b, step_embed, step):
    xf = x.reshape(_N, _H)
    c = (b[0] + step_embed[step, 0]).reshape(1, 1).astype(jnp.float32)

    g_flat, s_flat = pl.pallas_call(
        _score_body,
        grid=(_N // _BLK,),
        in_specs=[
            pl.BlockSpec((_BLK, _H), lambda i: (i, 0)),
            pl.BlockSpec((_H, 1), lambda i: (0, 0)),
            pl.BlockSpec((1, 1), lambda i: (0, 0)),
        ],
        out_specs=[
            pl.BlockSpec((_BLK, 1), lambda i: (i, 0)),
            pl.BlockSpec((_BLK, 1), lambda i: (i, 0)),
        ],
        out_shape=[
            jax.ShapeDtypeStruct((_N, 1), jnp.float32),
            jax.ShapeDtypeStruct((_N, 1), jnp.float32),
        ],
    )(xf, W, c)

    mask2d, aux = pl.pallas_call(
        _select_body,
        out_shape=[
            jax.ShapeDtypeStruct((_R, _R), jnp.float32),
            jax.ShapeDtypeStruct((1, 1), jnp.float32),
        ],
    )(s_flat.reshape(_R, _R))

    g = g_flat.reshape(_B, _S, 1)
    m = mask2d.reshape(_B, _S, 1)
    return g, m, aux.reshape(())


def kernel(x, W, b, step_embed, step):
    return _run(x, W, b, step_embed, step)
